# baseline (speedup 1.0000x reference)
import functools

import jax
import jax.numpy as jnp
from jax import lax
from jax.experimental import pallas as pl
from jax.experimental.pallas import tpu as pltpu
from jax.experimental.pallas import tpu_sc as plsc

N_LOCS = 10000
N_CLUSTERS = 10000
E = 320000
D = 128

NW = 32
CPW = 320
PADC = NW * CPW
CH = 12800
NCHUNK = E // CH
L = 16
SR = 256
NSLAB = 40
PADL = NSLAB * SR
BCAP = 64
REGCAP = 10016
OCAP = E + NSLAB * L * BCAP + 64
ARENA = 43008
CBASE = NSLAB * L * BCAP
RBASE = SR * D
DUMMY_DL = CPW
NEG_INF = float("-inf")

_mesh = plsc.VectorSubcoreMesh(core_axis_name="c", subcore_axis_name="s")


@functools.partial(
    pl.kernel,
    out_type=(
        jax.ShapeDtypeStruct((PADC, 2 * D), jnp.float32),
        jax.ShapeDtypeStruct((NW * NSLAB * REGCAP,), jnp.int32),
        jax.ShapeDtypeStruct((NW * OCAP,), jnp.int32),
    ),
    mesh=_mesh,
    compiler_params=pltpu.CompilerParams(needs_layout_passes=False),
    scratch_types=[
        pltpu.VMEM((CH,), jnp.int32),
        pltpu.VMEM((CH,), jnp.int32),
        pltpu.VMEM((ARENA,), jnp.int32),
        pltpu.VMEM((NSLAB * L,), jnp.int32),
        pltpu.VMEM((CPW + 1, D), jnp.float32),
        pltpu.VMEM((64, D), jnp.int32),
        pltpu.VMEM((64,), jnp.int32),
        pltpu.VMEM((64,), jnp.int32),
        pltpu.SMEM((NSLAB,), jnp.int32),
        pltpu.SMEM((8,), jnp.int32),
        pltpu.SemaphoreType.DMA,
    ],
)
def _loc2cluster_sc(src_hbm, dst_hbm, xlf_hbm, xl_hbm, xc_hbm,
                    out_hbm, recs_hbm, over_hbm,
                    dstc, srcc, arena, curs, acc, fbrows, idxb, fbrec,
                    hcur, misc, sem):
    wid = lax.axis_index("s") * 2 + lax.axis_index("c")
    lo = wid * CPW

    iota = lax.broadcasted_iota(jnp.int32, (L,), 0)
    zeros = jnp.zeros((L,), jnp.int32)
    neg = jnp.full((L,), NEG_INF, jnp.float32)

    def init_acc(r, _):
        for j in range(D // L):
            acc[r, pl.ds(j * L, L)] = neg
        return 0

    lax.fori_loop(0, CPW + 1, init_acc, 0)

    def init_curs(g, _):
        curs[pl.ds(g * L, L)] = zeros
        return 0

    lax.fori_loop(0, NSLAB, init_curs, 0)

    def init_hcur(s, _):
        hcur[s] = 0
        return 0

    lax.fori_loop(0, NSLAB, init_hcur, 0)
    misc[0] = 0

    dummy0 = jnp.full((L,), DUMMY_DL, jnp.int32)

    def init_cbase(g, _):
        arena[pl.ds(CBASE + g * L, L)] = dummy0
        return 0

    lax.fori_loop(0, (L * BCAP) // L, init_cbase, 0)

    def accum_rec(dl, row_ref, row_base):
        for j in range(D // L):
            sl = pl.ds(j * L, L)
            v = plsc.bitcast(row_ref[pl.ds(row_base + j * L, L)],
                             jnp.float32)
            acc[dl, sl] = jnp.maximum(acc[dl, sl], v)

    def fallback_block(src_ref, base):
        for t in range(4):
            rv = src_ref[pl.ds(base + t * L, L)]
            idxb[pl.ds(t * L, L)] = lax.shift_right_logical(rv, 9)
        pltpu.async_copy(xl_hbm.at[idxb], fbrows, sem).wait()
        for t in range(4):
            rv = src_ref[pl.ds(base + t * L, L)]
            for i in range(L):
                dl = rv[i] & 511
                for j in range(D // L):
                    sl = pl.ds(j * L, L)
                    v = plsc.bitcast(fbrows[t * L + i, sl], jnp.float32)
                    acc[dl, sl] = jnp.maximum(acc[dl, sl], v)

    def chunk_body(c, _):
        base_e = c * CH
        pltpu.sync_copy(dst_hbm.at[pl.ds(base_e, CH)], dstc)
        pltpu.sync_copy(src_hbm.at[pl.ds(base_e, CH)], srcc)

        def sel_body(g, _):
            d = dstc[pl.ds(g * L, L)]
            s = srcc[pl.ds(g * L, L)]
            m = (d >= lo) & (d < lo + CPW)
            nsel = plsc.all_reduce_population_count(m)

            @pl.when(nsel[0] > 0)
            def _bin():
                rec = lax.shift_left(s, 9) + (d - lo)
                sv = lax.shift_right_logical(s, 8)
                cidx = sv * L + iota
                cur = plsc.load_gather(curs, [cidx], mask=m)
                pos = sv * (L * BCAP) + iota * BCAP + cur
                plsc.store_scatter(arena, [pos], rec, mask=m)
                plsc.store_scatter(curs, [cidx], cur + 1, mask=m)
                fullv = jnp.where(((cur + 1) >= BCAP) & m, 1, 0)

                @pl.when(jnp.max(fullv) > 0)
                def _spill():
                    for i in range(L):
                        @pl.when(fullv[i] > 0)
                        def _one():
                            sl_i = sv[i]
                            sbase = sl_i * (L * BCAP) + i * BCAP
                            sc = misc[1]

                            @pl.when(sc >= 16)
                            def _drain_one():
                                pltpu.make_async_copy(
                                    over_hbm.at[pl.ds(wid * OCAP, BCAP)],
                                    arena.at[pl.ds(CBASE, BCAP)],
                                    sem).wait()

                            slot = pl.multiple_of(
                                CBASE + (sc % 16) * BCAP, 8)
                            for t in range(BCAP // L):
                                arena[pl.ds(slot + t * L, L)] = (
                                    arena[pl.ds(sbase + t * L, L)])
                            hc = hcur[sl_i]

                            @pl.when(hc + BCAP <= REGCAP)
                            def _fast():
                                pltpu.async_copy(
                                    arena.at[pl.ds(slot, BCAP)],
                                    recs_hbm.at[pl.ds(pl.multiple_of(
                                        wid * (NSLAB * REGCAP)
                                        + sl_i * REGCAP + hc, 8), BCAP)],
                                    sem)
                                hcur[sl_i] = hc + BCAP

                            @pl.when(hc + BCAP > REGCAP)
                            def _slow():
                                oc = misc[0]
                                pltpu.async_copy(
                                    arena.at[pl.ds(slot, BCAP)],
                                    over_hbm.at[pl.ds(pl.multiple_of(
                                        wid * OCAP + oc, 8), BCAP)],
                                    sem)
                                misc[0] = oc + BCAP

                            misc[1] = sc + 1
                            plsc.store_scatter(curs, [sv * L + iota],
                                               zeros, mask=(iota == i))
            return 0

        lax.fori_loop(0, CH // L, sel_body, 0)
        return 0

    misc[1] = 0
    lax.fori_loop(0, NCHUNK, chunk_body, 0)

    def drain_body(k, _):
        pltpu.make_async_copy(over_hbm.at[pl.ds(wid * OCAP, BCAP)],
                              arena.at[pl.ds(CBASE, BCAP)], sem).wait()
        return 0

    lax.fori_loop(0, jnp.minimum(misc[1], 16), drain_body, 0)

    def flush_slab(s, _):
        cv = curs[pl.ds(s * L, L)]
        dummy = lax.shift_left(s * SR, 9) + DUMMY_DL
        poff = jnp.int32(0)
        for i in range(L):
            cur_l = cv[i]
            len_l = ((cur_l + L - 1) // L) * L
            sbase = s * (L * BCAP) + i * BCAP
            for t in range(BCAP // L):
                posv = t * L + iota
                v = arena[pl.ds(sbase + t * L, L)]
                v = jnp.where(posv >= cur_l, dummy, v)

                @pl.when(t * L < len_l)
                def _cp():
                    arena[pl.ds(pl.multiple_of(CBASE + poff, 8)
                                + t * L, L)] = v
            poff = poff + len_l

        @pl.when(poff > 0)
        def _spill():
            hc = hcur[s]

            @pl.when(hc + L * BCAP <= REGCAP)
            def _fast():
                pltpu.sync_copy(arena.at[pl.ds(CBASE, L * BCAP)],
                                recs_hbm.at[pl.ds(pl.multiple_of(
                                    wid * (NSLAB * REGCAP)
                                    + s * REGCAP + hc, 8), L * BCAP)])
                hcur[s] = hc + poff

            @pl.when(hc + L * BCAP > REGCAP)
            def _slow():
                oc = misc[0]
                pltpu.sync_copy(arena.at[pl.ds(CBASE, L * BCAP)],
                                over_hbm.at[pl.ds(pl.multiple_of(
                                    wid * OCAP + oc, 8), L * BCAP)])
                misc[0] = oc + L * BCAP
        return 0

    lax.fori_loop(0, NSLAB, flush_slab, 0)

    def slab_body(s, _):
        pltpu.sync_copy(xlf_hbm.at[pl.ds(pl.multiple_of(s * SR * D, 8),
                                         SR * D)],
                        arena.at[pl.ds(0, SR * D)])
        pltpu.sync_copy(recs_hbm.at[pl.ds(pl.multiple_of(
                            wid * (NSLAB * REGCAP) + s * REGCAP, 8),
                            REGCAP)],
                        arena.at[pl.ds(RBASE, REGCAP)])
        cnt = hcur[s]

        def rec_group(g, _):
            rv = arena[pl.ds(RBASE + g * L, L)]
            for i in range(L):
                rec_i = rv[i]
                dl = rec_i & 511
                lsrc = lax.shift_right_logical(rec_i, 9) - s * SR
                accum_rec(dl, arena, lsrc * D)
            return 0

        lax.fori_loop(0, cnt // L, rec_group, 0)
        return 0

    lax.fori_loop(0, NSLAB, slab_body, 0)

    oc_total = misc[0]

    def over_body(b, _):
        pltpu.sync_copy(over_hbm.at[pl.ds(pl.multiple_of(
                            wid * OCAP + b * 64, 8), 64)], fbrec)
        fallback_block(fbrec, 0)
        return 0

    lax.fori_loop(0, oc_total // 64, over_body, 0)

    def flush_body(r, _):
        for j in range(D // L):
            sl = pl.ds(j * L, L)
            v = acc[r, sl]
            acc[r, sl] = jnp.where(v == NEG_INF, 0.0, v)
        return 0

    lax.fori_loop(0, CPW, flush_body, 0)
    pltpu.sync_copy(acc.at[pl.ds(0, CPW)],
                    out_hbm.at[pl.ds(lo, CPW), pl.ds(D, D)])
    pltpu.sync_copy(xc_hbm.at[pl.ds(lo, CPW)], acc.at[pl.ds(0, CPW)])
    pltpu.sync_copy(acc.at[pl.ds(0, CPW)],
                    out_hbm.at[pl.ds(lo, CPW), pl.ds(0, D)])


def kernel(x_locs, x_clusters, edge_index):
    ei = edge_index.astype(jnp.int32)
    src = ei[0]
    dst = ei[1]
    xc_pad = jnp.pad(x_clusters, ((0, PADC - N_CLUSTERS), (0, 0)))
    xl_pad = jax.lax.bitcast_convert_type(
        jnp.pad(x_locs, ((0, PADL - N_LOCS), (0, 0))), jnp.int32)
    xl_flat = xl_pad.reshape(PADL * D)
    out, _, _ = _loc2cluster_sc(src, dst, xl_flat, xl_pad, xc_pad)
    return out[:N_CLUSTERS]


# speedup vs baseline: 1.2759x; 1.2759x over previous
import functools

import jax
import jax.numpy as jnp
from jax import lax
from jax.experimental import pallas as pl
from jax.experimental.pallas import tpu as pltpu
from jax.experimental.pallas import tpu_sc as plsc

N_LOCS = 10000
N_CLUSTERS = 10000
E = 320000
D = 128

NW = 32
CPW = 320
PADC = NW * CPW
CH = 12800
NCHUNK = E // CH
L = 16
SR = 256
NSLAB = 40
PADL = NSLAB * SR
BCAP = 64
REGCAP = 10016
OCAP = E + NSLAB * L * BCAP + 64
ARENA = 43008
CBASE = NSLAB * L * BCAP
RBASE = SR * D
DUMMY_DL = CPW
NEG_INF = float("-inf")

_mesh = plsc.VectorSubcoreMesh(core_axis_name="c", subcore_axis_name="s")


@functools.partial(
    pl.kernel,
    out_type=(
        jax.ShapeDtypeStruct((PADC, 2 * D), jnp.float32),
        jax.ShapeDtypeStruct((NW * NSLAB * REGCAP,), jnp.int32),
        jax.ShapeDtypeStruct((NW * OCAP,), jnp.int32),
    ),
    mesh=_mesh,
    compiler_params=pltpu.CompilerParams(needs_layout_passes=False),
    scratch_types=[
        pltpu.VMEM((CH,), jnp.int32),
        pltpu.VMEM((CH,), jnp.int32),
        pltpu.VMEM((ARENA,), jnp.int32),
        pltpu.VMEM((NSLAB * L,), jnp.int32),
        pltpu.VMEM((CPW + 1, D), jnp.float32),
        pltpu.VMEM((64, D), jnp.int32),
        pltpu.VMEM((64,), jnp.int32),
        pltpu.VMEM((64,), jnp.int32),
        pltpu.SMEM((NSLAB,), jnp.int32),
        pltpu.SMEM((8,), jnp.int32),
        pltpu.SemaphoreType.DMA,
    ],
)
def _loc2cluster_sc(src_hbm, dst_hbm, xlf_hbm, xl_hbm, xc_hbm,
                    out_hbm, recs_hbm, over_hbm,
                    dstc, srcc, arena, curs, acc, fbrows, idxb, fbrec,
                    hcur, misc, sem):
    wid = lax.axis_index("s") * 2 + lax.axis_index("c")
    lo = wid * CPW

    iota = lax.broadcasted_iota(jnp.int32, (L,), 0)
    zeros = jnp.zeros((L,), jnp.int32)
    neg = jnp.full((L,), NEG_INF, jnp.float32)

    def init_acc(r, _):
        for j in range(D // L):
            acc[r, pl.ds(j * L, L)] = neg
        return 0

    lax.fori_loop(0, CPW + 1, init_acc, 0)

    def init_curs(g, _):
        curs[pl.ds(g * L, L)] = zeros
        return 0

    lax.fori_loop(0, NSLAB, init_curs, 0)

    def init_hcur(s, _):
        hcur[s] = 0
        return 0

    lax.fori_loop(0, NSLAB, init_hcur, 0)
    misc[0] = 0
    misc[1] = 0

    dummy0 = jnp.full((L,), DUMMY_DL, jnp.int32)

    def init_cbase(g, _):
        arena[pl.ds(CBASE + g * L, L)] = dummy0
        return 0

    lax.fori_loop(0, (L * BCAP) // L, init_cbase, 0)

    def accum_rec(dl, row_ref, row_base):
        for j in range(D // L):
            sl = pl.ds(j * L, L)
            v = plsc.bitcast(row_ref[pl.ds(row_base + j * L, L)],
                             jnp.float32)
            acc[dl, sl] = jnp.maximum(acc[dl, sl], v)

    def fallback_block(src_ref, base):
        for t in range(4):
            rv = src_ref[pl.ds(base + t * L, L)]
            idxb[pl.ds(t * L, L)] = lax.shift_right_logical(rv, 9)
        pltpu.async_copy(xl_hbm.at[idxb], fbrows, sem).wait()
        for t in range(4):
            rv = src_ref[pl.ds(base + t * L, L)]
            for i in range(L):
                dl = rv[i] & 511
                for j in range(D // L):
                    sl = pl.ds(j * L, L)
                    v = plsc.bitcast(fbrows[t * L + i, sl], jnp.float32)
                    acc[dl, sl] = jnp.maximum(acc[dl, sl], v)

    def chunk_body(c, _):
        base_e = c * CH
        pltpu.sync_copy(dst_hbm.at[pl.ds(base_e, CH)], dstc)
        pltpu.sync_copy(src_hbm.at[pl.ds(base_e, CH)], srcc)

        def sel_body(g, _):
            d = dstc[pl.ds(g * L, L)]
            s = srcc[pl.ds(g * L, L)]
            m = (d >= lo) & (d < lo + CPW)
            rec = lax.shift_left(s, 9) + (d - lo)
            sv = lax.shift_right_logical(s, 8)
            cidx = sv * L + iota
            cur = plsc.load_gather(curs, [cidx], mask=m)
            pos = sv * (L * BCAP) + iota * BCAP + cur
            plsc.store_scatter(arena, [pos], rec, mask=m)
            plsc.store_scatter(curs, [cidx], cur + 1, mask=m)
            fullv = jnp.where(((cur + 1) >= BCAP) & m, 1, 0)

            @pl.when(jnp.max(fullv) > 0)
            def _spill():
                for i in range(L):
                    @pl.when(fullv[i] > 0)
                    def _one():
                        sl_i = sv[i]
                        sbase = sl_i * (L * BCAP) + i * BCAP

                        sc = misc[1]

                        @pl.when(sc >= 16)
                        def _drain_one():
                            pltpu.make_async_copy(
                                over_hbm.at[pl.ds(wid * OCAP, BCAP)],
                                arena.at[pl.ds(CBASE, BCAP)],
                                sem).wait()

                        slot = pl.multiple_of(
                            CBASE + (sc % 16) * BCAP, 8)
                        for t in range(BCAP // L):
                            arena[pl.ds(slot + t * L, L)] = (
                                arena[pl.ds(sbase + t * L, L)])
                        hc2 = hcur[sl_i]

                        @pl.when(hc2 + BCAP <= REGCAP)
                        def _fast():
                            pltpu.async_copy(
                                arena.at[pl.ds(slot, BCAP)],
                                recs_hbm.at[pl.ds(pl.multiple_of(
                                    wid * (NSLAB * REGCAP)
                                    + sl_i * REGCAP + hc2, 8), BCAP)],
                                sem)
                            hcur[sl_i] = hc2 + BCAP

                        @pl.when(hc2 + BCAP > REGCAP)
                        def _slow():
                            oc = misc[0]
                            pltpu.async_copy(
                                arena.at[pl.ds(slot, BCAP)],
                                over_hbm.at[pl.ds(pl.multiple_of(
                                    wid * OCAP + oc, 8), BCAP)],
                                sem)
                            misc[0] = oc + BCAP

                        misc[1] = sc + 1

                        plsc.store_scatter(curs, [sv * L + iota], zeros,
                                           mask=(iota == i))
            return 0

        lax.fori_loop(0, CH // L, sel_body, 0)
        return 0

    lax.fori_loop(0, NCHUNK, chunk_body, 0)

    def drain_body(k, _):
        pltpu.make_async_copy(over_hbm.at[pl.ds(wid * OCAP, BCAP)],
                              arena.at[pl.ds(CBASE, BCAP)], sem).wait()
        return 0

    lax.fori_loop(0, jnp.minimum(misc[1], 16), drain_body, 0)

    def flush_slab(s, _):
        cv = curs[pl.ds(s * L, L)]
        dummy = lax.shift_left(s * SR, 9) + DUMMY_DL
        poff = jnp.int32(0)
        for i in range(L):
            cur_l = cv[i]
            len_l = ((cur_l + L - 1) // L) * L
            sbase = s * (L * BCAP) + i * BCAP
            for t in range(BCAP // L):
                posv = t * L + iota
                v = arena[pl.ds(sbase + t * L, L)]
                v = jnp.where(posv >= cur_l, dummy, v)

                @pl.when(t * L < len_l)
                def _cp():
                    arena[pl.ds(pl.multiple_of(CBASE + poff, 8)
                                + t * L, L)] = v
            poff = poff + len_l

        @pl.when(poff > 0)
        def _spill():
            hc = hcur[s]

            @pl.when(hc + L * BCAP <= REGCAP)
            def _fast():
                pltpu.sync_copy(arena.at[pl.ds(CBASE, L * BCAP)],
                                recs_hbm.at[pl.ds(pl.multiple_of(
                                    wid * (NSLAB * REGCAP)
                                    + s * REGCAP + hc, 8), L * BCAP)])
                hcur[s] = hc + poff

            @pl.when(hc + L * BCAP > REGCAP)
            def _slow():
                oc = misc[0]
                pltpu.sync_copy(arena.at[pl.ds(CBASE, L * BCAP)],
                                over_hbm.at[pl.ds(pl.multiple_of(
                                    wid * OCAP + oc, 8), L * BCAP)])
                misc[0] = oc + L * BCAP
        return 0

    lax.fori_loop(0, NSLAB, flush_slab, 0)

    def slab_body(s, _):
        pltpu.sync_copy(xlf_hbm.at[pl.ds(pl.multiple_of(s * SR * D, 8),
                                         SR * D)],
                        arena.at[pl.ds(0, SR * D)])
        pltpu.sync_copy(recs_hbm.at[pl.ds(pl.multiple_of(
                            wid * (NSLAB * REGCAP) + s * REGCAP, 8),
                            REGCAP)],
                        arena.at[pl.ds(RBASE, REGCAP)])
        cnt = hcur[s]

        def rec_group(g, _):
            rv = arena[pl.ds(RBASE + g * L, L)]
            for i in range(L):
                rec_i = rv[i]
                dl = rec_i & 511
                lsrc = lax.shift_right_logical(rec_i, 9) - s * SR
                accum_rec(dl, arena, lsrc * D)
            return 0

        lax.fori_loop(0, cnt // L, rec_group, 0)
        return 0

    lax.fori_loop(0, NSLAB, slab_body, 0)

    oc_total = misc[0]

    def over_body(b, _):
        pltpu.sync_copy(over_hbm.at[pl.ds(pl.multiple_of(
                            wid * OCAP + b * 64, 8), 64)], fbrec)
        fallback_block(fbrec, 0)
        return 0

    lax.fori_loop(0, oc_total // 64, over_body, 0)

    def flush_body(r, _):
        for j in range(D // L):
            sl = pl.ds(j * L, L)
            v = acc[r, sl]
            acc[r, sl] = jnp.where(v == NEG_INF, 0.0, v)
        return 0

    lax.fori_loop(0, CPW, flush_body, 0)
    pltpu.sync_copy(acc.at[pl.ds(0, CPW)],
                    out_hbm.at[pl.ds(lo, CPW), pl.ds(D, D)])
    pltpu.sync_copy(xc_hbm.at[pl.ds(lo, CPW)], acc.at[pl.ds(0, CPW)])
    pltpu.sync_copy(acc.at[pl.ds(0, CPW)],
                    out_hbm.at[pl.ds(lo, CPW), pl.ds(0, D)])


def kernel(x_locs, x_clusters, edge_index):
    ei = edge_index.astype(jnp.int32)
    src = ei[0]
    dst = ei[1]
    xc_pad = jnp.pad(x_clusters, ((0, PADC - N_CLUSTERS), (0, 0)))
    xl_pad = jax.lax.bitcast_convert_type(
        jnp.pad(x_locs, ((0, PADL - N_LOCS), (0, 0))), jnp.int32)
    xl_flat = xl_pad.reshape(PADL * D)
    out, _, _ = _loc2cluster_sc(src, dst, xl_flat, xl_pad, xc_pad)
    return out[:N_CLUSTERS]


# speedup vs baseline: 1.3310x; 1.0432x over previous
import functools

import jax
import jax.numpy as jnp
from jax import lax
from jax.experimental import pallas as pl
from jax.experimental.pallas import tpu as pltpu
from jax.experimental.pallas import tpu_sc as plsc

N_LOCS = 10000
N_CLUSTERS = 10000
E = 320000
D = 128

NW = 32
CPW = 320
PADC = NW * CPW
CH = 6400
NCHUNK = E // CH
L = 16
SR = 256
NSLAB = 40
PADL = NSLAB * SR
BCAP = 64
REGCAP = 5008
OCAP = E + NSLAB * L * BCAP + 64
ARENA = 43008
CBASE = NSLAB * L * BCAP
RBASE = SR * D
DUMMY_DL = CPW
NEG_INF = float("-inf")

_mesh = plsc.VectorSubcoreMesh(core_axis_name="c", subcore_axis_name="s")


@functools.partial(
    pl.kernel,
    out_type=(
        jax.ShapeDtypeStruct((PADC, 2 * D), jnp.float32),
        jax.ShapeDtypeStruct((NW * NSLAB * REGCAP,), jnp.int32),
        jax.ShapeDtypeStruct((NW * OCAP,), jnp.int32),
    ),
    mesh=_mesh,
    compiler_params=pltpu.CompilerParams(needs_layout_passes=False),
    scratch_types=[
        pltpu.VMEM((CH,), jnp.int32),
        pltpu.VMEM((CH,), jnp.int32),
        pltpu.VMEM((CH,), jnp.int32),
        pltpu.VMEM((CH,), jnp.int32),
        pltpu.VMEM((ARENA,), jnp.int32),
        pltpu.VMEM((NSLAB * L,), jnp.int32),
        pltpu.VMEM((CPW + 1, D), jnp.float32),
        pltpu.VMEM((64, D), jnp.int32),
        pltpu.VMEM((64,), jnp.int32),
        pltpu.VMEM((64,), jnp.int32),
        pltpu.SMEM((NSLAB,), jnp.int32),
        pltpu.SMEM((8,), jnp.int32),
        pltpu.SemaphoreType.DMA,
        pltpu.SemaphoreType.DMA,
    ],
)
def _loc2cluster_sc(src_hbm, dst_hbm, xlf_hbm, xl_hbm, xc_hbm,
                    out_hbm, recs_hbm, over_hbm,
                    dstcA, srccA, dstcB, srccB, arena, curs, acc,
                    fbrows, idxb, fbrec, hcur, misc, sem, semc):
    wid = lax.axis_index("s") * 2 + lax.axis_index("c")
    lo = wid * CPW

    iota = lax.broadcasted_iota(jnp.int32, (L,), 0)
    zeros = jnp.zeros((L,), jnp.int32)
    neg = jnp.full((L,), NEG_INF, jnp.float32)

    def init_acc(r, _):
        for j in range(D // L):
            acc[r, pl.ds(j * L, L)] = neg
        return 0

    lax.fori_loop(0, CPW + 1, init_acc, 0)

    def init_curs(g, _):
        curs[pl.ds(g * L, L)] = zeros
        return 0

    lax.fori_loop(0, NSLAB, init_curs, 0)

    def init_hcur(s, _):
        hcur[s] = 0
        return 0

    lax.fori_loop(0, NSLAB, init_hcur, 0)
    misc[0] = 0
    misc[1] = 0

    dummy0 = jnp.full((L,), DUMMY_DL, jnp.int32)

    def init_cbase(g, _):
        arena[pl.ds(CBASE + g * L, L)] = dummy0
        return 0

    lax.fori_loop(0, (L * BCAP) // L, init_cbase, 0)

    def accum_rec(dl, row_ref, row_base):
        for j in range(D // L):
            sl = pl.ds(j * L, L)
            v = plsc.bitcast(row_ref[pl.ds(row_base + j * L, L)],
                             jnp.float32)
            acc[dl, sl] = jnp.maximum(acc[dl, sl], v)

    def fallback_block(src_ref, base):
        for t in range(4):
            rv = src_ref[pl.ds(base + t * L, L)]
            idxb[pl.ds(t * L, L)] = lax.shift_right_logical(rv, 9)
        pltpu.async_copy(xl_hbm.at[idxb], fbrows, sem).wait()
        for t in range(4):
            rv = src_ref[pl.ds(base + t * L, L)]
            for i in range(L):
                dl = rv[i] & 511
                for j in range(D // L):
                    sl = pl.ds(j * L, L)
                    v = plsc.bitcast(fbrows[t * L + i, sl], jnp.float32)
                    acc[dl, sl] = jnp.maximum(acc[dl, sl], v)

    def fire_chunk(c, dbuf, sbuf):
        base_e = pl.multiple_of(c * CH, 8)
        pltpu.async_copy(dst_hbm.at[pl.ds(base_e, CH)], dbuf, semc)
        pltpu.async_copy(src_hbm.at[pl.ds(base_e, CH)], sbuf, semc)

    def drain_chunk(dbuf, sbuf):
        pltpu.make_async_copy(dst_hbm.at[pl.ds(0, CH)], dbuf, semc).wait()
        pltpu.make_async_copy(src_hbm.at[pl.ds(0, CH)], sbuf, semc).wait()

    def scan_chunk(dstc, srcc):
        def sel_body(g, _):
            d = dstc[pl.ds(g * L, L)]
            s = srcc[pl.ds(g * L, L)]
            m = (d >= lo) & (d < lo + CPW)
            rec = lax.shift_left(s, 9) + (d - lo)
            sv = lax.shift_right_logical(s, 8)
            cidx = sv * L + iota
            cur = plsc.load_gather(curs, [cidx], mask=m)
            pos = sv * (L * BCAP) + iota * BCAP + cur
            plsc.store_scatter(arena, [pos], rec, mask=m)
            plsc.store_scatter(curs, [cidx], cur + 1, mask=m)
            fullv = jnp.where(((cur + 1) >= BCAP) & m, 1, 0)

            @pl.when(jnp.max(fullv) > 0)
            def _spill():
                for i in range(L):
                    @pl.when(fullv[i] > 0)
                    def _one():
                        sl_i = sv[i]
                        sbase = sl_i * (L * BCAP) + i * BCAP

                        sc = misc[1]

                        @pl.when(sc >= 16)
                        def _drain_one():
                            pltpu.make_async_copy(
                                over_hbm.at[pl.ds(wid * OCAP, BCAP)],
                                arena.at[pl.ds(CBASE, BCAP)],
                                sem).wait()

                        slot = pl.multiple_of(
                            CBASE + (sc % 16) * BCAP, 8)
                        for t in range(BCAP // L):
                            arena[pl.ds(slot + t * L, L)] = (
                                arena[pl.ds(sbase + t * L, L)])
                        hc2 = hcur[sl_i]

                        @pl.when(hc2 + BCAP <= REGCAP)
                        def _fast():
                            pltpu.async_copy(
                                arena.at[pl.ds(slot, BCAP)],
                                recs_hbm.at[pl.ds(pl.multiple_of(
                                    wid * (NSLAB * REGCAP)
                                    + sl_i * REGCAP + hc2, 8), BCAP)],
                                sem)
                            hcur[sl_i] = hc2 + BCAP

                        @pl.when(hc2 + BCAP > REGCAP)
                        def _slow():
                            oc = misc[0]
                            pltpu.async_copy(
                                arena.at[pl.ds(slot, BCAP)],
                                over_hbm.at[pl.ds(pl.multiple_of(
                                    wid * OCAP + oc, 8), BCAP)],
                                sem)
                            misc[0] = oc + BCAP

                        misc[1] = sc + 1

                        plsc.store_scatter(curs, [sv * L + iota], zeros,
                                           mask=(iota == i))
            return 0

        lax.fori_loop(0, CH // L, sel_body, 0)

    def pair_body(k, _):
        fire_chunk(2 * k + 1, dstcB, srccB)
        drain_chunk(dstcA, srccA)
        scan_chunk(dstcA, srccA)

        @pl.when(2 * k + 2 < NCHUNK)
        def _next():
            fire_chunk(2 * k + 2, dstcA, srccA)

        drain_chunk(dstcB, srccB)
        scan_chunk(dstcB, srccB)
        return 0

    fire_chunk(0, dstcA, srccA)
    lax.fori_loop(0, NCHUNK // 2, pair_body, 0)

    def drain_body(k, _):
        pltpu.make_async_copy(over_hbm.at[pl.ds(wid * OCAP, BCAP)],
                              arena.at[pl.ds(CBASE, BCAP)], sem).wait()
        return 0

    lax.fori_loop(0, jnp.minimum(misc[1], 16), drain_body, 0)

    def flush_slab(s, _):
        cv = curs[pl.ds(s * L, L)]
        dummy = lax.shift_left(s * SR, 9) + DUMMY_DL
        poff = jnp.int32(0)
        for i in range(L):
            cur_l = cv[i]
            len_l = ((cur_l + L - 1) // L) * L
            sbase = s * (L * BCAP) + i * BCAP
            for t in range(BCAP // L):
                posv = t * L + iota
                v = arena[pl.ds(sbase + t * L, L)]
                v = jnp.where(posv >= cur_l, dummy, v)

                @pl.when(t * L < len_l)
                def _cp():
                    arena[pl.ds(pl.multiple_of(CBASE + poff, 8)
                                + t * L, L)] = v
            poff = poff + len_l

        @pl.when(poff > 0)
        def _spill():
            hc = hcur[s]

            @pl.when(hc + L * BCAP <= REGCAP)
            def _fast():
                pltpu.sync_copy(arena.at[pl.ds(CBASE, L * BCAP)],
                                recs_hbm.at[pl.ds(pl.multiple_of(
                                    wid * (NSLAB * REGCAP)
                                    + s * REGCAP + hc, 8), L * BCAP)])
                hcur[s] = hc + poff

            @pl.when(hc + L * BCAP > REGCAP)
            def _slow():
                oc = misc[0]
                pltpu.sync_copy(arena.at[pl.ds(CBASE, L * BCAP)],
                                over_hbm.at[pl.ds(pl.multiple_of(
                                    wid * OCAP + oc, 8), L * BCAP)])
                misc[0] = oc + L * BCAP
        return 0

    lax.fori_loop(0, NSLAB, flush_slab, 0)

    def slab_body(s, _):
        pltpu.sync_copy(xlf_hbm.at[pl.ds(pl.multiple_of(s * SR * D, 8),
                                         SR * D)],
                        arena.at[pl.ds(0, SR * D)])
        pltpu.sync_copy(recs_hbm.at[pl.ds(pl.multiple_of(
                            wid * (NSLAB * REGCAP) + s * REGCAP, 8),
                            REGCAP)],
                        arena.at[pl.ds(RBASE, REGCAP)])
        cnt = hcur[s]

        def rec_group(g, _):
            rv = arena[pl.ds(RBASE + g * L, L)]
            for i in range(L):
                rec_i = rv[i]
                dl = rec_i & 511
                lsrc = lax.shift_right_logical(rec_i, 9) - s * SR
                accum_rec(dl, arena, lsrc * D)
            return 0

        lax.fori_loop(0, cnt // L, rec_group, 0)
        return 0

    lax.fori_loop(0, NSLAB, slab_body, 0)

    oc_total = misc[0]

    def over_body(b, _):
        pltpu.sync_copy(over_hbm.at[pl.ds(pl.multiple_of(
                            wid * OCAP + b * 64, 8), 64)], fbrec)
        fallback_block(fbrec, 0)
        return 0

    lax.fori_loop(0, oc_total // 64, over_body, 0)

    def flush_body(r, _):
        for j in range(D // L):
            sl = pl.ds(j * L, L)
            v = acc[r, sl]
            acc[r, sl] = jnp.where(v == NEG_INF, 0.0, v)
        return 0

    lax.fori_loop(0, CPW, flush_body, 0)
    pltpu.sync_copy(acc.at[pl.ds(0, CPW)],
                    out_hbm.at[pl.ds(lo, CPW), pl.ds(D, D)])
    pltpu.sync_copy(xc_hbm.at[pl.ds(lo, CPW)], acc.at[pl.ds(0, CPW)])
    pltpu.sync_copy(acc.at[pl.ds(0, CPW)],
                    out_hbm.at[pl.ds(lo, CPW), pl.ds(0, D)])


def kernel(x_locs, x_clusters, edge_index):
    ei = edge_index.astype(jnp.int32)
    src = ei[0]
    dst = ei[1]
    xc_pad = jnp.pad(x_clusters, ((0, PADC - N_CLUSTERS), (0, 0)))
    xl_pad = jax.lax.bitcast_convert_type(
        jnp.pad(x_locs, ((0, PADL - N_LOCS), (0, 0))), jnp.int32)
    xl_flat = xl_pad.reshape(PADL * D)
    out, _, _ = _loc2cluster_sc(src, dst, xl_flat, xl_pad, xc_pad)
    return out[:N_CLUSTERS]


# speedup vs baseline: 1.9235x; 1.4451x over previous
import functools

import jax
import jax.numpy as jnp
from jax import lax
from jax.experimental import pallas as pl
from jax.experimental.pallas import tpu as pltpu
from jax.experimental.pallas import tpu_sc as plsc

N_LOCS = 10000
N_CLUSTERS = 10000
E = 320000
D = 128

NW = 32
CPW = 320
PADC = NW * CPW
CH = 6400
NCHUNK = E // CH
L = 16
SR = 256
NSLAB = 40
PADL = NSLAB * SR
BCAP = 64
REGCAP = 5008
OCAP = E + NSLAB * L * BCAP + 64
ARENA = 43008
CBASE = NSLAB * L * BCAP
RBASE = SR * D
DUMMY_DL = CPW
NEG_INF = float("-inf")

_mesh = plsc.VectorSubcoreMesh(core_axis_name="c", subcore_axis_name="s")


@functools.partial(
    pl.kernel,
    out_type=(
        jax.ShapeDtypeStruct((PADC, 2 * D), jnp.float32),
        jax.ShapeDtypeStruct((NW * NSLAB * REGCAP,), jnp.int32),
        jax.ShapeDtypeStruct((NW * OCAP,), jnp.int32),
    ),
    mesh=_mesh,
    compiler_params=pltpu.CompilerParams(needs_layout_passes=False),
    scratch_types=[
        pltpu.VMEM((CH,), jnp.int32),
        pltpu.VMEM((CH,), jnp.int32),
        pltpu.VMEM((CH,), jnp.int32),
        pltpu.VMEM((CH,), jnp.int32),
        pltpu.VMEM((CH,), jnp.int32),
        pltpu.VMEM((ARENA,), jnp.int32),
        pltpu.VMEM((NSLAB * L,), jnp.int32),
        pltpu.VMEM((CPW + 1, D), jnp.float32),
        pltpu.VMEM((64, D), jnp.int32),
        pltpu.VMEM((64,), jnp.int32),
        pltpu.VMEM((64,), jnp.int32),
        pltpu.SMEM((NSLAB,), jnp.int32),
        pltpu.SMEM((8,), jnp.int32),
        pltpu.SemaphoreType.DMA,
        pltpu.SemaphoreType.DMA,
    ],
)
def _loc2cluster_sc(src_hbm, dst_hbm, xlf_hbm, xl_hbm, xc_hbm,
                    out_hbm, recs_hbm, over_hbm,
                    dstcA, srccA, dstcB, srccB, selrec, arena, curs, acc,
                    fbrows, idxb, fbrec, hcur, misc, sem, semc):
    wid = lax.axis_index("s") * 2 + lax.axis_index("c")
    lo = wid * CPW

    iota = lax.broadcasted_iota(jnp.int32, (L,), 0)
    zeros = jnp.zeros((L,), jnp.int32)
    neg = jnp.full((L,), NEG_INF, jnp.float32)

    def init_acc(r, _):
        for j in range(D // L):
            acc[r, pl.ds(j * L, L)] = neg
        return 0

    lax.fori_loop(0, CPW + 1, init_acc, 0)

    def init_curs(g, _):
        curs[pl.ds(g * L, L)] = zeros
        return 0

    lax.fori_loop(0, NSLAB, init_curs, 0)

    def init_hcur(s, _):
        hcur[s] = 0
        return 0

    lax.fori_loop(0, NSLAB, init_hcur, 0)
    misc[0] = 0
    misc[1] = 0

    dummy0 = jnp.full((L,), DUMMY_DL, jnp.int32)

    def init_cbase(g, _):
        arena[pl.ds(CBASE + g * L, L)] = dummy0
        return 0

    lax.fori_loop(0, (L * BCAP) // L, init_cbase, 0)

    def accum_rec(dl, row_ref, row_base):
        for j in range(D // L):
            sl = pl.ds(j * L, L)
            v = plsc.bitcast(row_ref[pl.ds(row_base + j * L, L)],
                             jnp.float32)
            acc[dl, sl] = jnp.maximum(acc[dl, sl], v)

    def fallback_block(src_ref, base):
        for t in range(4):
            rv = src_ref[pl.ds(base + t * L, L)]
            idxb[pl.ds(t * L, L)] = lax.shift_right_logical(rv, 9)
        pltpu.async_copy(xl_hbm.at[idxb], fbrows, sem).wait()
        for t in range(4):
            rv = src_ref[pl.ds(base + t * L, L)]
            for i in range(L):
                dl = rv[i] & 511
                for j in range(D // L):
                    sl = pl.ds(j * L, L)
                    v = plsc.bitcast(fbrows[t * L + i, sl], jnp.float32)
                    acc[dl, sl] = jnp.maximum(acc[dl, sl], v)

    def fire_chunk(c, dbuf, sbuf):
        base_e = pl.multiple_of(c * CH, 8)
        pltpu.async_copy(dst_hbm.at[pl.ds(base_e, CH)], dbuf, semc)
        pltpu.async_copy(src_hbm.at[pl.ds(base_e, CH)], sbuf, semc)

    def drain_chunk(dbuf, sbuf):
        pltpu.make_async_copy(dst_hbm.at[pl.ds(0, CH)], dbuf, semc).wait()
        pltpu.make_async_copy(src_hbm.at[pl.ds(0, CH)], sbuf, semc).wait()

    def scan_chunk(dstc, srcc):
        def sel_body(g, n):
            d = dstc[pl.ds(g * L, L)]
            s = srcc[pl.ds(g * L, L)]
            m = (d >= lo) & (d < lo + CPW)
            rec = lax.shift_left(s, 9) + (d - lo)
            mcum = plsc.cumsum(jnp.where(m, 1, 0))
            cpos = n + mcum - 1
            plsc.store_scatter(selrec, [cpos], rec, mask=m)
            return n + mcum[L - 1]

        nsel = lax.fori_loop(0, CH // L, sel_body, jnp.int32(0))

        def bin_body(g, _):
            rec = selrec[pl.ds(g * L, L)]
            m = (g * L + iota) < nsel
            sv = lax.shift_right_logical(rec, 17)
            cidx = sv * L + iota
            cur = plsc.load_gather(curs, [cidx], mask=m)
            pos = sv * (L * BCAP) + iota * BCAP + cur
            plsc.store_scatter(arena, [pos], rec, mask=m)
            plsc.store_scatter(curs, [cidx], cur + 1, mask=m)
            fullv = jnp.where(((cur + 1) >= BCAP) & m, 1, 0)

            @pl.when(jnp.max(fullv) > 0)
            def _spill():
                for i in range(L):
                    @pl.when(fullv[i] > 0)
                    def _one():
                        sl_i = sv[i]
                        sbase = sl_i * (L * BCAP) + i * BCAP

                        sc = misc[1]

                        @pl.when(sc >= 16)
                        def _drain_one():
                            pltpu.make_async_copy(
                                over_hbm.at[pl.ds(wid * OCAP, BCAP)],
                                arena.at[pl.ds(CBASE, BCAP)],
                                sem).wait()

                        slot = pl.multiple_of(
                            CBASE + (sc % 16) * BCAP, 8)
                        for t in range(BCAP // L):
                            arena[pl.ds(slot + t * L, L)] = (
                                arena[pl.ds(sbase + t * L, L)])
                        hc2 = hcur[sl_i]

                        @pl.when(hc2 + BCAP <= REGCAP)
                        def _fast():
                            pltpu.async_copy(
                                arena.at[pl.ds(slot, BCAP)],
                                recs_hbm.at[pl.ds(pl.multiple_of(
                                    wid * (NSLAB * REGCAP)
                                    + sl_i * REGCAP + hc2, 8), BCAP)],
                                sem)
                            hcur[sl_i] = hc2 + BCAP

                        @pl.when(hc2 + BCAP > REGCAP)
                        def _slow():
                            oc = misc[0]
                            pltpu.async_copy(
                                arena.at[pl.ds(slot, BCAP)],
                                over_hbm.at[pl.ds(pl.multiple_of(
                                    wid * OCAP + oc, 8), BCAP)],
                                sem)
                            misc[0] = oc + BCAP

                        misc[1] = sc + 1

                        plsc.store_scatter(curs, [sv * L + iota], zeros,
                                           mask=(iota == i))
            return 0

        lax.fori_loop(0, (nsel + L - 1) // L, bin_body, 0)

    def pair_body(k, _):
        fire_chunk(2 * k + 1, dstcB, srccB)
        drain_chunk(dstcA, srccA)
        scan_chunk(dstcA, srccA)

        @pl.when(2 * k + 2 < NCHUNK)
        def _next():
            fire_chunk(2 * k + 2, dstcA, srccA)

        drain_chunk(dstcB, srccB)
        scan_chunk(dstcB, srccB)
        return 0

    fire_chunk(0, dstcA, srccA)
    lax.fori_loop(0, NCHUNK // 2, pair_body, 0)

    def drain_body(k, _):
        pltpu.make_async_copy(over_hbm.at[pl.ds(wid * OCAP, BCAP)],
                              arena.at[pl.ds(CBASE, BCAP)], sem).wait()
        return 0

    lax.fori_loop(0, jnp.minimum(misc[1], 16), drain_body, 0)

    def flush_slab(s, _):
        cv = curs[pl.ds(s * L, L)]
        dummy = lax.shift_left(s * SR, 9) + DUMMY_DL
        poff = jnp.int32(0)
        for i in range(L):
            cur_l = cv[i]
            len_l = ((cur_l + L - 1) // L) * L
            sbase = s * (L * BCAP) + i * BCAP
            for t in range(BCAP // L):
                posv = t * L + iota
                v = arena[pl.ds(sbase + t * L, L)]
                v = jnp.where(posv >= cur_l, dummy, v)

                @pl.when(t * L < len_l)
                def _cp():
                    arena[pl.ds(pl.multiple_of(CBASE + poff, 8)
                                + t * L, L)] = v
            poff = poff + len_l

        @pl.when(poff > 0)
        def _spill():
            hc = hcur[s]

            @pl.when(hc + L * BCAP <= REGCAP)
            def _fast():
                pltpu.sync_copy(arena.at[pl.ds(CBASE, L * BCAP)],
                                recs_hbm.at[pl.ds(pl.multiple_of(
                                    wid * (NSLAB * REGCAP)
                                    + s * REGCAP + hc, 8), L * BCAP)])
                hcur[s] = hc + poff

            @pl.when(hc + L * BCAP > REGCAP)
            def _slow():
                oc = misc[0]
                pltpu.sync_copy(arena.at[pl.ds(CBASE, L * BCAP)],
                                over_hbm.at[pl.ds(pl.multiple_of(
                                    wid * OCAP + oc, 8), L * BCAP)])
                misc[0] = oc + L * BCAP
        return 0

    lax.fori_loop(0, NSLAB, flush_slab, 0)

    def slab_body(s, _):
        pltpu.sync_copy(xlf_hbm.at[pl.ds(pl.multiple_of(s * SR * D, 8),
                                         SR * D)],
                        arena.at[pl.ds(0, SR * D)])
        pltpu.sync_copy(recs_hbm.at[pl.ds(pl.multiple_of(
                            wid * (NSLAB * REGCAP) + s * REGCAP, 8),
                            REGCAP)],
                        arena.at[pl.ds(RBASE, REGCAP)])
        cnt = hcur[s]

        def rec_group(g, _):
            rv = arena[pl.ds(RBASE + g * L, L)]
            for i in range(L):
                rec_i = rv[i]
                dl = rec_i & 511
                lsrc = lax.shift_right_logical(rec_i, 9) - s * SR
                accum_rec(dl, arena, lsrc * D)
            return 0

        lax.fori_loop(0, cnt // L, rec_group, 0)
        return 0

    lax.fori_loop(0, NSLAB, slab_body, 0)

    oc_total = misc[0]

    def over_body(b, _):
        pltpu.sync_copy(over_hbm.at[pl.ds(pl.multiple_of(
                            wid * OCAP + b * 64, 8), 64)], fbrec)
        fallback_block(fbrec, 0)
        return 0

    lax.fori_loop(0, oc_total // 64, over_body, 0)

    def flush_body(r, _):
        for j in range(D // L):
            sl = pl.ds(j * L, L)
            v = acc[r, sl]
            acc[r, sl] = jnp.where(v == NEG_INF, 0.0, v)
        return 0

    lax.fori_loop(0, CPW, flush_body, 0)
    pltpu.sync_copy(acc.at[pl.ds(0, CPW)],
                    out_hbm.at[pl.ds(lo, CPW), pl.ds(D, D)])
    pltpu.sync_copy(xc_hbm.at[pl.ds(lo, CPW)], acc.at[pl.ds(0, CPW)])
    pltpu.sync_copy(acc.at[pl.ds(0, CPW)],
                    out_hbm.at[pl.ds(lo, CPW), pl.ds(0, D)])


def kernel(x_locs, x_clusters, edge_index):
    ei = edge_index.astype(jnp.int32)
    src = ei[0]
    dst = ei[1]
    xc_pad = jnp.pad(x_clusters, ((0, PADC - N_CLUSTERS), (0, 0)))
    xl_pad = jax.lax.bitcast_convert_type(
        jnp.pad(x_locs, ((0, PADL - N_LOCS), (0, 0))), jnp.int32)
    xl_flat = xl_pad.reshape(PADL * D)
    out, _, _ = _loc2cluster_sc(src, dst, xl_flat, xl_pad, xc_pad)
    return out[:N_CLUSTERS]
